# baseline (speedup 1.0000x reference)
import dataclasses
import functools

import jax
import jax.numpy as jnp
from jax import lax
from jax.experimental import pallas as pl
from jax.experimental.pallas import tpu as pltpu
from jax.experimental.pallas import tpu_sc as plsc

DIM = 64
BATCH = 16384
PAD_DIM = 128
NUM_CORES = 2
NUM_SUBCORES = 16
NUM_WORKERS = NUM_CORES * NUM_SUBCORES
ROWS_PER_WORKER = BATCH // NUM_WORKERS
STAGE_ROWS = 128
NUM_STAGES = ROWS_PER_WORKER // STAGE_ROWS
LANES = 16
GROUPS_PER_STAGE = STAGE_ROWS // LANES


TR_COLS = 512
TR_GRID = -(-100000 // TR_COLS)


def _relayout_kernel(dt_ref, out_ref):
  x = dt_ref[...].T
  out_ref[...] = jnp.concatenate(
      [x, jnp.zeros((TR_COLS, PAD_DIM - DIM), jnp.float32)], axis=1)


def _pack_diag(dt):
  return pl.pallas_call(
      _relayout_kernel,
      grid=(TR_GRID,),
      in_specs=[pl.BlockSpec((DIM, TR_COLS), lambda i: (0, i))],
      out_specs=pl.BlockSpec((TR_COLS, PAD_DIM), lambda i: (i, 0)),
      out_shape=jax.ShapeDtypeStruct((100000, PAD_DIM), jnp.float32),
  )(dt)


def _sc_kernel(diagp_hbm, idx_hbm, ht_hbm, tt_hbm, out_hbm,
               idx_v, rel0, rel1, ht_v, tt_v, out_v,
               sem_h, sem_t, sem_g0, sem_g1):
  wid = lax.axis_index("s") * NUM_CORES + lax.axis_index("c")
  base = wid * ROWS_PER_WORKER

  pltpu.sync_copy(idx_hbm.at[wid], idx_v)
  copy_h = pltpu.async_copy(
      ht_hbm.at[:, pl.ds(base, ROWS_PER_WORKER)], ht_v, sem_h)
  copy_t = pltpu.async_copy(
      tt_hbm.at[:, pl.ds(base, ROWS_PER_WORKER)], tt_v, sem_t)

  rel = (rel0, rel1)
  sems = (sem_g0, sem_g1)
  iota = lax.iota(jnp.int32, LANES)

  def compute_stage(s, relbuf):
    @pl.loop(0, GROUPS_PER_STAGE)
    def _(g):
      bsl = pl.ds(s * STAGE_ROWS + g * LANES, LANES)
      rows = g * LANES + iota
      acc = None
      for d in range(DIM):
        hh = ht_v[d, bsl] * tt_v[d, bsl]
        rcol = plsc.load_gather(relbuf, [rows, jnp.full((LANES,), d, jnp.int32)])
        acc = hh * rcol if acc is None else acc + hh * rcol
      out_v[bsl] = acc

  gathers = [None] * NUM_STAGES
  gathers[0] = pltpu.async_copy(diagp_hbm.at[idx_v.at[0]], rel[0], sems[0])
  copy_h.wait()
  copy_t.wait()
  for s in range(NUM_STAGES):
    if s + 1 < NUM_STAGES:
      gathers[s + 1] = pltpu.async_copy(
          diagp_hbm.at[idx_v.at[s + 1]], rel[(s + 1) % 2], sems[(s + 1) % 2])
    gathers[s].wait()
    compute_stage(s, rel[s % 2])

  pltpu.sync_copy(out_v, out_hbm.at[pl.ds(base, ROWS_PER_WORKER)])


@jax.jit
def _dist_mult(h, r, t, diag):
  idx = r.astype(jnp.int32).reshape(NUM_WORKERS, NUM_STAGES, STAGE_ROWS)
  diagp = _pack_diag(diag.T)
  ht = h.T
  tt = t.T
  mesh = plsc.VectorSubcoreMesh(core_axis_name="c", subcore_axis_name="s")
  cp = pltpu.CompilerParams()
  for field, value in (("needs_layout_passes", False),
                       ("use_tc_tiling_on_sc", True)):
    if field in pltpu.CompilerParams.__dataclass_fields__:
      cp = dataclasses.replace(cp, **{field: value})
  run = pl.kernel(
      _sc_kernel,
      out_type=jax.ShapeDtypeStruct((BATCH,), jnp.float32),
      mesh=mesh,
      compiler_params=cp,
      scratch_types=[
          pltpu.VMEM((NUM_STAGES, STAGE_ROWS), jnp.int32),
          pltpu.VMEM((STAGE_ROWS, PAD_DIM), jnp.float32),
          pltpu.VMEM((STAGE_ROWS, PAD_DIM), jnp.float32),
          pltpu.VMEM((DIM, ROWS_PER_WORKER), jnp.float32),
          pltpu.VMEM((DIM, ROWS_PER_WORKER), jnp.float32),
          pltpu.VMEM((ROWS_PER_WORKER,), jnp.float32),
          pltpu.SemaphoreType.DMA,
          pltpu.SemaphoreType.DMA,
          pltpu.SemaphoreType.DMA,
          pltpu.SemaphoreType.DMA,
      ],
  )
  return run(diagp, idx, ht, tt)


def kernel(h, r, t, diag):
  return _dist_mult(h, r, t, diag)


# speedup vs baseline: 2.0286x; 2.0286x over previous
import dataclasses
import functools

import jax
import jax.numpy as jnp
from jax import lax
from jax.experimental import pallas as pl
from jax.experimental.pallas import tpu as pltpu
from jax.experimental.pallas import tpu_sc as plsc

DIM = 64
BATCH = 16384
PAD_DIM = 128
NUM_REL = 100000
NUM_CORES = 2
NUM_SUBCORES = 16
NUM_WORKERS = NUM_CORES * NUM_SUBCORES
ROWS_PER_WORKER = BATCH // NUM_WORKERS
STAGE_ROWS = 128
NUM_STAGES = ROWS_PER_WORKER // STAGE_ROWS
LANES = 16
GROUPS_PER_STAGE = STAGE_ROWS // LANES

TR_COLS = 4096
TR_GRID = -(-NUM_REL // TR_COLS)


def _relayout_kernel(dt_ref, out_ref):
  x = dt_ref[...].T
  out_ref[...] = jnp.concatenate(
      [x, jnp.zeros((TR_COLS, PAD_DIM - DIM), jnp.float32)], axis=1)


def _pack_diag(dt):
  return pl.pallas_call(
      _relayout_kernel,
      grid=(TR_GRID,),
      in_specs=[pl.BlockSpec((DIM, TR_COLS), lambda i: (0, i))],
      out_specs=pl.BlockSpec((TR_COLS, PAD_DIM), lambda i: (i, 0)),
      out_shape=jax.ShapeDtypeStruct((NUM_REL, PAD_DIM), jnp.float32),
  )(dt)


MUL_COLS = 4096


def _mul_kernel(ht_ref, tt_ref, out_ref):
  out_ref[...] = ht_ref[...] * tt_ref[...]


def _mul_ht(ht, tt):
  return pl.pallas_call(
      _mul_kernel,
      grid=(BATCH // MUL_COLS,),
      in_specs=[pl.BlockSpec((DIM, MUL_COLS), lambda i: (0, i)),
                pl.BlockSpec((DIM, MUL_COLS), lambda i: (0, i))],
      out_specs=pl.BlockSpec((DIM, MUL_COLS), lambda i: (0, i)),
      out_shape=jax.ShapeDtypeStruct((DIM, BATCH), jnp.float32),
  )(ht, tt)


def _sc_kernel(diagp_hbm, idx_hbm, p_hbm, out_hbm,
               idx_v, rel0, rel1, p_v, out_v,
               sem_p, sem_g0, sem_g1):
  wid = lax.axis_index("s") * NUM_CORES + lax.axis_index("c")
  base = wid * ROWS_PER_WORKER

  pltpu.sync_copy(idx_hbm.at[wid], idx_v)
  copy_p = pltpu.async_copy(
      p_hbm.at[:, pl.ds(base, ROWS_PER_WORKER)], p_v, sem_p)

  rel = (rel0, rel1)
  sems = (sem_g0, sem_g1)
  iota = lax.iota(jnp.int32, LANES)

  def compute_stage(s, relbuf):
    @pl.loop(0, GROUPS_PER_STAGE)
    def _(g):
      bsl = pl.ds(s * STAGE_ROWS + g * LANES, LANES)
      rows = g * LANES + iota
      acc = [None] * 4
      for d in range(DIM):
        rcol = plsc.load_gather(
            relbuf, [rows, jnp.full((LANES,), d, jnp.int32)])
        term = p_v[d, bsl] * rcol
        k = d % 4
        acc[k] = term if acc[k] is None else acc[k] + term
      out_v[bsl] = (acc[0] + acc[1]) + (acc[2] + acc[3])

  gathers = [None] * NUM_STAGES
  gathers[0] = pltpu.async_copy(diagp_hbm.at[idx_v.at[0]], rel[0], sems[0])
  copy_p.wait()
  for s in range(NUM_STAGES):
    if s + 1 < NUM_STAGES:
      gathers[s + 1] = pltpu.async_copy(
          diagp_hbm.at[idx_v.at[s + 1]], rel[(s + 1) % 2], sems[(s + 1) % 2])
    gathers[s].wait()
    compute_stage(s, rel[s % 2])

  pltpu.sync_copy(out_v, out_hbm.at[pl.ds(base, ROWS_PER_WORKER)])


@jax.jit
def _dist_mult(h, r, t, diag):
  idx = r.astype(jnp.int32).reshape(NUM_WORKERS, NUM_STAGES, STAGE_ROWS)
  diagp = _pack_diag(diag.T)
  p = _mul_ht(h.T, t.T)
  mesh = plsc.VectorSubcoreMesh(core_axis_name="c", subcore_axis_name="s")
  cp = pltpu.CompilerParams()
  for field, value in (("needs_layout_passes", False),
                       ("use_tc_tiling_on_sc", True)):
    if field in pltpu.CompilerParams.__dataclass_fields__:
      cp = dataclasses.replace(cp, **{field: value})
  run = pl.kernel(
      _sc_kernel,
      out_type=jax.ShapeDtypeStruct((BATCH,), jnp.float32),
      mesh=mesh,
      compiler_params=cp,
      scratch_types=[
          pltpu.VMEM((NUM_STAGES, STAGE_ROWS), jnp.int32),
          pltpu.VMEM((STAGE_ROWS, PAD_DIM), jnp.float32),
          pltpu.VMEM((STAGE_ROWS, PAD_DIM), jnp.float32),
          pltpu.VMEM((DIM, ROWS_PER_WORKER), jnp.float32),
          pltpu.VMEM((ROWS_PER_WORKER,), jnp.float32),
          pltpu.SemaphoreType.DMA,
          pltpu.SemaphoreType.DMA,
          pltpu.SemaphoreType.DMA,
      ],
  )
  return run(diagp, idx, p)


def kernel(h, r, t, diag):
  return _dist_mult(h, r, t, diag)


# speedup vs baseline: 2.0388x; 1.0050x over previous
import dataclasses
import functools

import jax
import jax.numpy as jnp
from jax import lax
from jax.experimental import pallas as pl
from jax.experimental.pallas import tpu as pltpu
from jax.experimental.pallas import tpu_sc as plsc

DIM = 64
BATCH = 16384
PAD_DIM = 128
NUM_REL = 100000
NUM_CORES = 2
NUM_SUBCORES = 16
NUM_WORKERS = NUM_CORES * NUM_SUBCORES
ROWS_PER_WORKER = BATCH // NUM_WORKERS
STAGE_ROWS = 128
NUM_STAGES = ROWS_PER_WORKER // STAGE_ROWS
LANES = 16
GROUPS_PER_STAGE = STAGE_ROWS // LANES

TR_COLS = 8192
TR_GRID = -(-NUM_REL // TR_COLS)


def _relayout_kernel(dt_ref, out_ref):
  x = dt_ref[...].T
  out_ref[...] = jnp.concatenate(
      [x, jnp.zeros((TR_COLS, PAD_DIM - DIM), jnp.float32)], axis=1)


def _pack_diag(dt):
  return pl.pallas_call(
      _relayout_kernel,
      grid=(TR_GRID,),
      in_specs=[pl.BlockSpec((DIM, TR_COLS), lambda i: (0, i))],
      out_specs=pl.BlockSpec((TR_COLS, PAD_DIM), lambda i: (i, 0)),
      out_shape=jax.ShapeDtypeStruct((NUM_REL, PAD_DIM), jnp.float32),
  )(dt)


def _sc_kernel(diagp_hbm, idx_hbm, ht_hbm, tt_hbm, out_hbm,
               idx_v, rel0, rel1, ht_v, tt_v, out_v,
               sem_h, sem_t, sem_g0, sem_g1):
  wid = lax.axis_index("s") * NUM_CORES + lax.axis_index("c")
  base = wid * ROWS_PER_WORKER

  pltpu.sync_copy(idx_hbm.at[wid], idx_v)
  copy_h = pltpu.async_copy(
      ht_hbm.at[:, pl.ds(base, ROWS_PER_WORKER)], ht_v, sem_h)
  copy_t = pltpu.async_copy(
      tt_hbm.at[:, pl.ds(base, ROWS_PER_WORKER)], tt_v, sem_t)

  rel = (rel0, rel1)
  sems = (sem_g0, sem_g1)
  iota = lax.iota(jnp.int32, LANES)

  def compute_stage(s, relbuf):
    @pl.loop(0, GROUPS_PER_STAGE)
    def _(g):
      bsl = pl.ds(s * STAGE_ROWS + g * LANES, LANES)
      rows = g * LANES + iota
      acc = [None] * 4
      for d in range(DIM):
        rcol = plsc.load_gather(
            relbuf, [rows, jnp.full((LANES,), d, jnp.int32)])
        term = ht_v[d, bsl] * tt_v[d, bsl] * rcol
        k = d % 4
        acc[k] = term if acc[k] is None else acc[k] + term
      out_v[bsl] = (acc[0] + acc[1]) + (acc[2] + acc[3])

  gathers = [None] * NUM_STAGES
  gathers[0] = pltpu.async_copy(diagp_hbm.at[idx_v.at[0]], rel[0], sems[0])
  copy_h.wait()
  copy_t.wait()
  for s in range(NUM_STAGES):
    if s + 1 < NUM_STAGES:
      gathers[s + 1] = pltpu.async_copy(
          diagp_hbm.at[idx_v.at[s + 1]], rel[(s + 1) % 2], sems[(s + 1) % 2])
    gathers[s].wait()
    compute_stage(s, rel[s % 2])

  pltpu.sync_copy(out_v, out_hbm.at[pl.ds(base, ROWS_PER_WORKER)])


@jax.jit
def _dist_mult(h, r, t, diag):
  idx = r.astype(jnp.int32).reshape(NUM_WORKERS, NUM_STAGES, STAGE_ROWS)
  diagp = _pack_diag(diag.T)
  mesh = plsc.VectorSubcoreMesh(core_axis_name="c", subcore_axis_name="s")
  cp = pltpu.CompilerParams()
  for field, value in (("needs_layout_passes", False),
                       ("use_tc_tiling_on_sc", True)):
    if field in pltpu.CompilerParams.__dataclass_fields__:
      cp = dataclasses.replace(cp, **{field: value})
  run = pl.kernel(
      _sc_kernel,
      out_type=jax.ShapeDtypeStruct((BATCH,), jnp.float32),
      mesh=mesh,
      compiler_params=cp,
      scratch_types=[
          pltpu.VMEM((NUM_STAGES, STAGE_ROWS), jnp.int32),
          pltpu.VMEM((STAGE_ROWS, PAD_DIM), jnp.float32),
          pltpu.VMEM((STAGE_ROWS, PAD_DIM), jnp.float32),
          pltpu.VMEM((DIM, ROWS_PER_WORKER), jnp.float32),
          pltpu.VMEM((DIM, ROWS_PER_WORKER), jnp.float32),
          pltpu.VMEM((ROWS_PER_WORKER,), jnp.float32),
          pltpu.SemaphoreType.DMA,
          pltpu.SemaphoreType.DMA,
          pltpu.SemaphoreType.DMA,
          pltpu.SemaphoreType.DMA,
      ],
  )
  return run(diagp, idx, h.T, t.T)


def kernel(h, r, t, diag):
  return _dist_mult(h, r, t, diag)


# speedup vs baseline: 2.5395x; 1.2456x over previous
import dataclasses
import functools

import jax
import jax.numpy as jnp
from jax import lax
from jax.experimental import pallas as pl
from jax.experimental.pallas import tpu as pltpu
from jax.experimental.pallas import tpu_sc as plsc

DIM = 64
BATCH = 16384
PAD_DIM = 128
NUM_REL = 100000
NUM_CORES = 2
NUM_SUBCORES = 16
NUM_WORKERS = NUM_CORES * NUM_SUBCORES
ROWS_PER_WORKER = BATCH // NUM_WORKERS
STAGE_ROWS = 128
NUM_STAGES = ROWS_PER_WORKER // STAGE_ROWS
LANES = 16
DIM_CHUNKS = DIM // LANES
GROUPS_PER_STAGE = STAGE_ROWS // LANES

TR_COLS = 8192


def _pack_diag_kernel(dt_ref, out_ref):
  x = dt_ref[...].T
  out_ref[...] = jnp.concatenate(
      [x, jnp.zeros((TR_COLS, PAD_DIM - DIM), jnp.float32)], axis=1)


def _pack_diag(dt):
  return pl.pallas_call(
      _pack_diag_kernel,
      grid=(-(-NUM_REL // TR_COLS),),
      in_specs=[pl.BlockSpec((DIM, TR_COLS), lambda i: (0, i))],
      out_specs=pl.BlockSpec((TR_COLS, PAD_DIM), lambda i: (i, 0)),
      out_shape=jax.ShapeDtypeStruct((NUM_REL, PAD_DIM), jnp.float32),
  )(dt)


def _pack_q_kernel(ht_ref, tt_ref, out_ref):
  x = (ht_ref[...] * tt_ref[...]).T
  out_ref[...] = jnp.concatenate(
      [x, jnp.zeros((TR_COLS, PAD_DIM - DIM), jnp.float32)], axis=1)


def _pack_q(ht, tt):
  return pl.pallas_call(
      _pack_q_kernel,
      grid=(BATCH // TR_COLS,),
      in_specs=[pl.BlockSpec((DIM, TR_COLS), lambda i: (0, i)),
                pl.BlockSpec((DIM, TR_COLS), lambda i: (0, i))],
      out_specs=pl.BlockSpec((TR_COLS, PAD_DIM), lambda i: (i, 0)),
      out_shape=jax.ShapeDtypeStruct((BATCH, PAD_DIM), jnp.float32),
  )(ht, tt)


def _sc_kernel(diagp_hbm, idx_hbm, q_hbm, out_hbm,
               idx_v, rel0, rel1, q_v, out_v,
               sem_q, sem_g0, sem_g1):
  wid = lax.axis_index("s") * NUM_CORES + lax.axis_index("c")
  base = wid * ROWS_PER_WORKER

  pltpu.sync_copy(idx_hbm.at[wid], idx_v)
  copy_q = pltpu.async_copy(
      q_hbm.at[pl.ds(base, ROWS_PER_WORKER)], q_v, sem_q)

  rel = (rel0, rel1)
  sems = (sem_g0, sem_g1)
  lane = lax.iota(jnp.int32, LANES)

  def compute_stage(s, relbuf):
    @pl.loop(0, GROUPS_PER_STAGE)
    def _(g):
      res = [jnp.zeros((LANES,), jnp.float32) for _ in range(2)]
      for k in range(LANES):
        li = g * LANES + k
        qrow = s * STAGE_ROWS + g * LANES + k
        acc = None
        for c in range(DIM_CHUNKS):
          term = (q_v[qrow, pl.ds(c * LANES, LANES)]
                  * relbuf[li, pl.ds(c * LANES, LANES)])
          acc = term if acc is None else acc + term
        ch = k & 1
        res[ch] = jnp.where(lane == k, jnp.sum(acc), res[ch])
      out_v[pl.ds(s * STAGE_ROWS + g * LANES, LANES)] = res[0] + res[1]

  gathers = [None] * NUM_STAGES
  gathers[0] = pltpu.async_copy(diagp_hbm.at[idx_v.at[0]], rel[0], sems[0])
  copy_q.wait()
  for s in range(NUM_STAGES):
    if s + 1 < NUM_STAGES:
      gathers[s + 1] = pltpu.async_copy(
          diagp_hbm.at[idx_v.at[s + 1]], rel[(s + 1) % 2], sems[(s + 1) % 2])
    gathers[s].wait()
    compute_stage(s, rel[s % 2])

  pltpu.sync_copy(out_v, out_hbm.at[pl.ds(base, ROWS_PER_WORKER)])


@jax.jit
def _dist_mult(h, r, t, diag):
  idx = r.astype(jnp.int32).reshape(NUM_WORKERS, NUM_STAGES, STAGE_ROWS)
  diagp = _pack_diag(diag.T)
  q = _pack_q(h.T, t.T)
  mesh = plsc.VectorSubcoreMesh(core_axis_name="c", subcore_axis_name="s")
  cp = pltpu.CompilerParams()
  for field, value in (("needs_layout_passes", False),
                       ("use_tc_tiling_on_sc", True)):
    if field in pltpu.CompilerParams.__dataclass_fields__:
      cp = dataclasses.replace(cp, **{field: value})
  run = pl.kernel(
      _sc_kernel,
      out_type=jax.ShapeDtypeStruct((BATCH,), jnp.float32),
      mesh=mesh,
      compiler_params=cp,
      scratch_types=[
          pltpu.VMEM((NUM_STAGES, STAGE_ROWS), jnp.int32),
          pltpu.VMEM((STAGE_ROWS, PAD_DIM), jnp.float32),
          pltpu.VMEM((STAGE_ROWS, PAD_DIM), jnp.float32),
          pltpu.VMEM((ROWS_PER_WORKER, PAD_DIM), jnp.float32),
          pltpu.VMEM((ROWS_PER_WORKER,), jnp.float32),
          pltpu.SemaphoreType.DMA,
          pltpu.SemaphoreType.DMA,
          pltpu.SemaphoreType.DMA,
      ],
  )
  return run(diagp, idx, q)


def kernel(h, r, t, diag):
  return _dist_mult(h, r, t, diag)
